# baseline (speedup 1.0000x reference)
import functools

import jax
import jax.numpy as jnp
from jax import lax
from jax.experimental import pallas as pl
from jax.experimental.pallas import tpu as pltpu
from jax.experimental.pallas import tpu_sc as plsc

N = 10000
D = 128
E = 320000

_info = plsc.get_sparse_core_info()
NC = _info.num_cores
NS = _info.num_subcores
NW = NC * NS
CH = 50
NCH = E // NW // CH
GB = 40
NG = NCH // GB
NBUF = 4
RPT = 624
ZR = 16

_mesh = plsc.VectorSubcoreMesh(core_axis_name="c", subcore_axis_name="s")


@functools.partial(
    pl.kernel,
    out_type=jax.ShapeDtypeStruct((NC, N, D), jnp.float32),
    mesh=_mesh,
    scratch_types=[
        pltpu.VMEM((GB, CH), jnp.int32),
        pltpu.VMEM((GB, CH), jnp.int32),
        pltpu.VMEM((NBUF, CH, D), jnp.float32),
        pltpu.VMEM((ZR, D), jnp.float32),
        pltpu.VMEM_SHARED((N, D), jnp.float32),
        [pltpu.SemaphoreType.DMA] * NBUF,
        [pltpu.SemaphoreType.DMA] * NBUF,
    ],
)
def _edge_agg(h_hbm, src_hbm, dst_hbm, out_hbm, src_b, dst_b, rows, zero_v,
              acc, sem_g, sem_s):
    c = lax.axis_index("c")
    s = lax.axis_index("s")
    wid = s * NC + c

    zvec = jnp.zeros((16,), jnp.float32)
    for r in range(ZR):
        for k in range(D // 16):
            zero_v[r, pl.ds(k * 16, 16)] = zvec

    def zero_issue(i, carry):
        pltpu.async_copy(zero_v, acc.at[pl.ds(s * RPT + i * ZR, ZR)], sem_g[0])
        return carry

    lax.fori_loop(0, RPT // ZR, zero_issue, 0)

    @pl.when(s == NS - 1)
    def _():
        pltpu.async_copy(zero_v, acc.at[pl.ds(NS * RPT, ZR)], sem_g[0])

    def zero_drain(i, carry):
        pltpu.make_async_copy(zero_v, acc.at[pl.ds(0, ZR)], sem_g[0]).wait()
        return carry

    lax.fori_loop(0, RPT // ZR, zero_drain, 0)

    @pl.when(s == NS - 1)
    def _():
        pltpu.make_async_copy(zero_v, acc.at[pl.ds(0, ZR)], sem_g[0]).wait()

    plsc.subcore_barrier()

    def _wait_gather(b):
        pltpu.make_async_copy(h_hbm.at[src_b.at[0]], rows.at[b], sem_g[b]).wait()

    def _wait_scatter(b):
        pltpu.make_async_copy(rows.at[b], acc.at[dst_b.at[0]], sem_s[b]).wait()

    def group_body(k, carry):
        pltpu.sync_copy(src_hbm.at[wid, pl.ds(k * GB, GB)], src_b)
        pltpu.sync_copy(dst_hbm.at[wid, pl.ds(k * GB, GB)], dst_b)
        pltpu.async_copy(h_hbm.at[src_b.at[0]], rows.at[0], sem_g[0])
        pltpu.async_copy(h_hbm.at[src_b.at[1]], rows.at[1], sem_g[1])

        def edge_body(g, carry2):
            for b in range(NBUF):
                j = NBUF * g + b
                b2 = (b + 2) % NBUF
                _wait_gather(b)
                pltpu.async_copy(rows.at[b], acc.at[dst_b.at[j]], sem_s[b],
                                 add=True)

                @pl.when(j >= 2)
                def _():
                    _wait_scatter(b2)

                @pl.when(j <= GB - 3)
                def _():
                    pltpu.async_copy(h_hbm.at[src_b.at[j + 2]], rows.at[b2],
                                     sem_g[b2])
            return carry2

        lax.fori_loop(0, GB // NBUF, edge_body, 0)
        _wait_scatter((GB - 2) % NBUF)
        _wait_scatter((GB - 1) % NBUF)
        return carry

    lax.fori_loop(0, NG, group_body, 0)
    plsc.subcore_barrier()

    pltpu.sync_copy(acc.at[pl.ds(s * RPT, RPT)],
                    out_hbm.at[c, pl.ds(s * RPT, RPT)])

    @pl.when(s == NS - 1)
    def _():
        pltpu.sync_copy(acc.at[pl.ds(NS * RPT, N - NS * RPT)],
                        out_hbm.at[c, pl.ds(NS * RPT, N - NS * RPT)])


_R = 1000


def _relu_sum_mm_kernel(p_ref, w_ref, o_ref):
    h = p_ref[0] + p_ref[1]
    o_ref[...] = jnp.maximum(
        jnp.dot(h, w_ref[...], preferred_element_type=jnp.float32,
                precision=jax.lax.Precision.HIGHEST), 0.0)


def _relu_sum_mm(p, W):
    return pl.pallas_call(
        _relu_sum_mm_kernel,
        grid=(N // _R,),
        in_specs=[pl.BlockSpec((NC, _R, D), lambda i: (0, i, 0)),
                  pl.BlockSpec((D, D), lambda i: (0, 0))],
        out_specs=pl.BlockSpec((_R, D), lambda i: (i, 0)),
        out_shape=jax.ShapeDtypeStruct((N, D), jnp.float32),
    )(p, W)


def kernel(x, edge_index, W1, W2):
    src = edge_index[0].reshape(NW, NCH, CH)
    dst = edge_index[1].reshape(NW, NCH, CH)
    p1 = _edge_agg(x, src, dst)
    h = _relu_sum_mm(p1, W1)
    p2 = _edge_agg(h, src, dst)
    return _relu_sum_mm(p2, W2)


# speedup vs baseline: 1.1474x; 1.1474x over previous
import functools

import jax
import jax.numpy as jnp
from jax import lax
from jax.experimental import pallas as pl
from jax.experimental.pallas import tpu as pltpu
from jax.experimental.pallas import tpu_sc as plsc

N = 10000
D = 128
E = 320000

_info = plsc.get_sparse_core_info()
NC = _info.num_cores
NS = _info.num_subcores
NW = NC * NS
CH = 125
NCH = E // NW // CH
GB = 16
NG = NCH // GB
NBUF = 2
RPT = 624
ZR = 16

_mesh = plsc.VectorSubcoreMesh(core_axis_name="c", subcore_axis_name="s")


@functools.partial(
    pl.kernel,
    out_type=jax.ShapeDtypeStruct((NC, N, D), jnp.float32),
    mesh=_mesh,
    scratch_types=[
        pltpu.VMEM((GB, CH), jnp.int32),
        pltpu.VMEM((GB, CH), jnp.int32),
        pltpu.VMEM((NBUF, CH, D), jnp.float32),
        pltpu.VMEM((ZR, D), jnp.float32),
        pltpu.VMEM_SHARED((N, D), jnp.float32),
        [pltpu.SemaphoreType.DMA] * NBUF,
    ],
)
def _edge_agg(h_hbm, src_hbm, dst_hbm, out_hbm, src_b, dst_b, rows, zero_v,
              acc, sem_g):
    c = lax.axis_index("c")
    s = lax.axis_index("s")
    wid = s * NC + c

    zvec = jnp.zeros((16,), jnp.float32)
    for r in range(ZR):
        for k in range(D // 16):
            zero_v[r, pl.ds(k * 16, 16)] = zvec

    def zero_issue(i, carry):
        pltpu.async_copy(zero_v, acc.at[pl.ds(s * RPT + i * ZR, ZR)], sem_g[0])
        return carry

    lax.fori_loop(0, RPT // ZR, zero_issue, 0)

    @pl.when(s == NS - 1)
    def _():
        pltpu.async_copy(zero_v, acc.at[pl.ds(NS * RPT, ZR)], sem_g[0])

    def zero_drain(i, carry):
        pltpu.make_async_copy(zero_v, acc.at[pl.ds(0, ZR)], sem_g[0]).wait()
        return carry

    lax.fori_loop(0, RPT // ZR, zero_drain, 0)

    @pl.when(s == NS - 1)
    def _():
        pltpu.make_async_copy(zero_v, acc.at[pl.ds(0, ZR)], sem_g[0]).wait()

    plsc.subcore_barrier()

    def _wait_gather(b):
        pltpu.make_async_copy(h_hbm.at[src_b.at[0]], rows.at[b], sem_g[b]).wait()

    def group_body(k, carry):
        pltpu.sync_copy(src_hbm.at[wid, pl.ds(k * GB, GB)], src_b)
        pltpu.sync_copy(dst_hbm.at[wid, pl.ds(k * GB, GB)], dst_b)
        pltpu.async_copy(h_hbm.at[src_b.at[0]], rows.at[0], sem_g[0])

        def edge_body(g, carry2):
            j = 2 * g
            pltpu.async_copy(h_hbm.at[src_b.at[j + 1]], rows.at[1], sem_g[1])
            _wait_gather(0)
            pltpu.sync_copy(rows.at[0], acc.at[dst_b.at[j]], add=True)

            @pl.when(g < GB // 2 - 1)
            def _():
                pltpu.async_copy(h_hbm.at[src_b.at[j + 2]], rows.at[0],
                                 sem_g[0])

            _wait_gather(1)
            pltpu.sync_copy(rows.at[1], acc.at[dst_b.at[j + 1]], add=True)
            return carry2

        lax.fori_loop(0, GB // 2, edge_body, 0)
        return carry

    lax.fori_loop(0, NG, group_body, 0)
    plsc.subcore_barrier()

    pltpu.sync_copy(acc.at[pl.ds(s * RPT, RPT)],
                    out_hbm.at[c, pl.ds(s * RPT, RPT)])

    @pl.when(s == NS - 1)
    def _():
        pltpu.sync_copy(acc.at[pl.ds(NS * RPT, N - NS * RPT)],
                        out_hbm.at[c, pl.ds(NS * RPT, N - NS * RPT)])


_R = 1000


def _relu_sum_mm_kernel(p_ref, w_ref, o_ref):
    h = p_ref[0] + p_ref[1]
    o_ref[...] = jnp.maximum(
        jnp.dot(h, w_ref[...], preferred_element_type=jnp.float32,
                precision=jax.lax.Precision.HIGHEST), 0.0)


def _relu_sum_mm(p, W):
    return pl.pallas_call(
        _relu_sum_mm_kernel,
        grid=(N // _R,),
        in_specs=[pl.BlockSpec((NC, _R, D), lambda i: (0, i, 0)),
                  pl.BlockSpec((D, D), lambda i: (0, 0))],
        out_specs=pl.BlockSpec((_R, D), lambda i: (i, 0)),
        out_shape=jax.ShapeDtypeStruct((N, D), jnp.float32),
    )(p, W)


def kernel(x, edge_index, W1, W2):
    src = edge_index[0].reshape(NW, NCH, CH)
    dst = edge_index[1].reshape(NW, NCH, CH)
    p1 = _edge_agg(x, src, dst)
    h = _relu_sum_mm(p1, W1)
    p2 = _edge_agg(h, src, dst)
    return _relu_sum_mm(p2, W2)


# speedup vs baseline: 1.1965x; 1.0428x over previous
import functools

import jax
import jax.numpy as jnp
from jax import lax
from jax.experimental import pallas as pl
from jax.experimental.pallas import tpu as pltpu
from jax.experimental.pallas import tpu_sc as plsc

N = 10000
D = 128
E = 320000

_info = plsc.get_sparse_core_info()
NC = _info.num_cores
NS = _info.num_subcores
NW = NC * NS
CH = 125
NCH = E // NW // CH
GB = 16
NG = NCH // GB
NBUF = 2
RPT = 624
ZR = 16

_mesh = plsc.VectorSubcoreMesh(core_axis_name="c", subcore_axis_name="s")


@functools.partial(
    pl.kernel,
    out_type=jax.ShapeDtypeStruct((NC, N, D), jnp.float32),
    mesh=_mesh,
    scratch_types=[
        pltpu.VMEM((2, GB, CH), jnp.int32),
        pltpu.VMEM((2, GB, CH), jnp.int32),
        pltpu.VMEM((NBUF, CH, D), jnp.float32),
        pltpu.VMEM((ZR, D), jnp.float32),
        pltpu.VMEM_SHARED((N, D), jnp.float32),
        [pltpu.SemaphoreType.DMA] * NBUF,
        pltpu.SemaphoreType.DMA,
    ],
)
def _edge_agg(h_hbm, src_hbm, dst_hbm, out_hbm, src_b, dst_b, rows, zero_v,
              acc, sem_g, sem_i):
    c = lax.axis_index("c")
    s = lax.axis_index("s")
    wid = s * NC + c

    zvec = jnp.zeros((16,), jnp.float32)
    for r in range(ZR):
        for k in range(D // 16):
            zero_v[r, pl.ds(k * 16, 16)] = zvec

    def zero_issue(i, carry):
        pltpu.async_copy(zero_v, acc.at[pl.ds(s * RPT + i * ZR, ZR)], sem_g[0])
        return carry

    lax.fori_loop(0, RPT // ZR, zero_issue, 0)

    @pl.when(s == NS - 1)
    def _():
        pltpu.async_copy(zero_v, acc.at[pl.ds(NS * RPT, ZR)], sem_g[0])

    def zero_drain(i, carry):
        pltpu.make_async_copy(zero_v, acc.at[pl.ds(0, ZR)], sem_g[0]).wait()
        return carry

    lax.fori_loop(0, RPT // ZR, zero_drain, 0)

    @pl.when(s == NS - 1)
    def _():
        pltpu.make_async_copy(zero_v, acc.at[pl.ds(0, ZR)], sem_g[0]).wait()

    plsc.subcore_barrier()

    def _wait_gather(b):
        pltpu.make_async_copy(h_hbm.at[src_b.at[0, 0]], rows.at[b],
                              sem_g[b]).wait()

    def _wait_idx():
        pltpu.make_async_copy(src_hbm.at[0, pl.ds(0, GB)], src_b.at[0],
                              sem_i).wait()

    pltpu.async_copy(src_hbm.at[wid, pl.ds(0, GB)], src_b.at[0], sem_i)
    pltpu.async_copy(dst_hbm.at[wid, pl.ds(0, GB)], dst_b.at[0], sem_i)
    _wait_idx()
    _wait_idx()

    def group_body(k, carry):
        k2 = lax.rem(k, 2)
        kn = lax.rem(k + 1, 2)

        @pl.when(k < NG - 1)
        def _():
            off = pl.multiple_of((k + 1) * GB, 8)
            pltpu.async_copy(src_hbm.at[wid, pl.ds(off, GB)], src_b.at[kn],
                             sem_i)
            pltpu.async_copy(dst_hbm.at[wid, pl.ds(off, GB)], dst_b.at[kn],
                             sem_i)

        pltpu.async_copy(h_hbm.at[src_b.at[k2, 0]], rows.at[0], sem_g[0])

        def edge_body(g, carry2):
            j = 2 * g
            pltpu.async_copy(h_hbm.at[src_b.at[k2, j + 1]], rows.at[1],
                             sem_g[1])
            _wait_gather(0)
            pltpu.sync_copy(rows.at[0], acc.at[dst_b.at[k2, j]], add=True)

            @pl.when(g < GB // 2 - 1)
            def _():
                pltpu.async_copy(h_hbm.at[src_b.at[k2, j + 2]], rows.at[0],
                                 sem_g[0])

            _wait_gather(1)
            pltpu.sync_copy(rows.at[1], acc.at[dst_b.at[k2, j + 1]], add=True)
            return carry2

        lax.fori_loop(0, GB // 2, edge_body, 0)

        @pl.when(k < NG - 1)
        def _():
            _wait_idx()
            _wait_idx()

        return carry

    lax.fori_loop(0, NG, group_body, 0)
    plsc.subcore_barrier()

    pltpu.sync_copy(acc.at[pl.ds(s * RPT, RPT)],
                    out_hbm.at[c, pl.ds(s * RPT, RPT)])

    @pl.when(s == NS - 1)
    def _():
        pltpu.sync_copy(acc.at[pl.ds(NS * RPT, N - NS * RPT)],
                        out_hbm.at[c, pl.ds(NS * RPT, N - NS * RPT)])


_R = 1000


def _relu_sum_mm_kernel(p_ref, w_ref, o_ref):
    h = p_ref[0] + p_ref[1]
    o_ref[...] = jnp.maximum(
        jnp.dot(h, w_ref[...], preferred_element_type=jnp.float32,
                precision=jax.lax.Precision.HIGHEST), 0.0)


def _relu_sum_mm(p, W):
    return pl.pallas_call(
        _relu_sum_mm_kernel,
        grid=(N // _R,),
        in_specs=[pl.BlockSpec((NC, _R, D), lambda i: (0, i, 0)),
                  pl.BlockSpec((D, D), lambda i: (0, 0))],
        out_specs=pl.BlockSpec((_R, D), lambda i: (i, 0)),
        out_shape=jax.ShapeDtypeStruct((N, D), jnp.float32),
    )(p, W)


def kernel(x, edge_index, W1, W2):
    src = edge_index[0].reshape(NW, NCH, CH)
    dst = edge_index[1].reshape(NW, NCH, CH)
    p1 = _edge_agg(x, src, dst)
    h = _relu_sum_mm(p1, W1)
    p2 = _edge_agg(h, src, dst)
    return _relu_sum_mm(p2, W2)


# speedup vs baseline: 1.2755x; 1.0660x over previous
import functools

import jax
import jax.numpy as jnp
from jax import lax
from jax.experimental import pallas as pl
from jax.experimental.pallas import tpu as pltpu
from jax.experimental.pallas import tpu_sc as plsc

N = 10000
D = 128
E = 320000

_info = plsc.get_sparse_core_info()
NC = _info.num_cores
NS = _info.num_subcores
NW = NC * NS
CH = 125
NCH = E // NW // CH
GB = 16
NG = NCH // GB
NBUF = 2
RPT = 624
ZR = 16

BDT = jnp.bfloat16

_mesh = plsc.VectorSubcoreMesh(core_axis_name="c", subcore_axis_name="s")


@functools.partial(
    pl.kernel,
    out_type=jax.ShapeDtypeStruct((NC, N, D), BDT),
    mesh=_mesh,
    scratch_types=[
        pltpu.VMEM((2, GB, CH), jnp.int32),
        pltpu.VMEM((2, GB, CH), jnp.int32),
        pltpu.VMEM((NBUF, CH, D), BDT),
        pltpu.VMEM((ZR, D), BDT),
        pltpu.VMEM_SHARED((N, D), BDT),
        [pltpu.SemaphoreType.DMA] * NBUF,
        pltpu.SemaphoreType.DMA,
    ],
    compiler_params=pltpu.CompilerParams(use_tc_tiling_on_sc=False),
)
def _edge_agg(h_hbm, src_hbm, dst_hbm, out_hbm, src_b, dst_b, rows, zero_v,
              acc, sem_g, sem_i):
    c = lax.axis_index("c")
    s = lax.axis_index("s")
    wid = s * NC + c

    zvec = jnp.zeros((32,), BDT)
    for r in range(ZR):
        for k in range(D // 32):
            zero_v[r, pl.ds(k * 32, 32)] = zvec

    def zero_issue(i, carry):
        pltpu.async_copy(zero_v, acc.at[pl.ds(s * RPT + i * ZR, ZR)], sem_g[0])
        return carry

    lax.fori_loop(0, RPT // ZR, zero_issue, 0)

    @pl.when(s == NS - 1)
    def _():
        pltpu.async_copy(zero_v, acc.at[pl.ds(NS * RPT, ZR)], sem_g[0])

    def zero_drain(i, carry):
        pltpu.make_async_copy(zero_v, acc.at[pl.ds(0, ZR)], sem_g[0]).wait()
        return carry

    lax.fori_loop(0, RPT // ZR, zero_drain, 0)

    @pl.when(s == NS - 1)
    def _():
        pltpu.make_async_copy(zero_v, acc.at[pl.ds(0, ZR)], sem_g[0]).wait()

    plsc.subcore_barrier()

    def _wait_gather(b):
        pltpu.make_async_copy(h_hbm.at[src_b.at[0, 0]], rows.at[b],
                              sem_g[b]).wait()

    def _wait_idx():
        pltpu.make_async_copy(src_hbm.at[0, pl.ds(0, GB)], src_b.at[0],
                              sem_i).wait()

    pltpu.async_copy(src_hbm.at[wid, pl.ds(0, GB)], src_b.at[0], sem_i)
    pltpu.async_copy(dst_hbm.at[wid, pl.ds(0, GB)], dst_b.at[0], sem_i)
    _wait_idx()
    _wait_idx()

    def group_body(k, carry):
        k2 = lax.rem(k, 2)
        kn = lax.rem(k + 1, 2)

        @pl.when(k < NG - 1)
        def _():
            off = pl.multiple_of((k + 1) * GB, 8)
            pltpu.async_copy(src_hbm.at[wid, pl.ds(off, GB)], src_b.at[kn],
                             sem_i)
            pltpu.async_copy(dst_hbm.at[wid, pl.ds(off, GB)], dst_b.at[kn],
                             sem_i)

        pltpu.async_copy(h_hbm.at[src_b.at[k2, 0]], rows.at[0], sem_g[0])

        def edge_body(g, carry2):
            j = 2 * g
            pltpu.async_copy(h_hbm.at[src_b.at[k2, j + 1]], rows.at[1],
                             sem_g[1])
            _wait_gather(0)
            pltpu.sync_copy(rows.at[0], acc.at[dst_b.at[k2, j]], add=True)

            @pl.when(g < GB // 2 - 1)
            def _():
                pltpu.async_copy(h_hbm.at[src_b.at[k2, j + 2]], rows.at[0],
                                 sem_g[0])

            _wait_gather(1)
            pltpu.sync_copy(rows.at[1], acc.at[dst_b.at[k2, j + 1]], add=True)
            return carry2

        lax.fori_loop(0, GB // 2, edge_body, 0)

        @pl.when(k < NG - 1)
        def _():
            _wait_idx()
            _wait_idx()

        return carry

    lax.fori_loop(0, NG, group_body, 0)
    plsc.subcore_barrier()

    pltpu.sync_copy(acc.at[pl.ds(s * RPT, RPT)],
                    out_hbm.at[c, pl.ds(s * RPT, RPT)])

    @pl.when(s == NS - 1)
    def _():
        pltpu.sync_copy(acc.at[pl.ds(NS * RPT, N - NS * RPT)],
                        out_hbm.at[c, pl.ds(NS * RPT, N - NS * RPT)])


_R = 1000


def _relu_sum_mm_kernel(p_ref, w_ref, o_ref):
    h = p_ref[0].astype(jnp.float32) + p_ref[1].astype(jnp.float32)
    r = jnp.maximum(
        jnp.dot(h, w_ref[...], preferred_element_type=jnp.float32,
                precision=jax.lax.Precision.HIGHEST), 0.0)
    o_ref[...] = r.astype(o_ref.dtype)


def _relu_sum_mm(p, W, out_dtype):
    return pl.pallas_call(
        _relu_sum_mm_kernel,
        grid=(N // _R,),
        in_specs=[pl.BlockSpec((NC, _R, D), lambda i: (0, i, 0)),
                  pl.BlockSpec((D, D), lambda i: (0, 0))],
        out_specs=pl.BlockSpec((_R, D), lambda i: (i, 0)),
        out_shape=jax.ShapeDtypeStruct((N, D), out_dtype),
    )(p, W)


def kernel(x, edge_index, W1, W2):
    src = edge_index[0].reshape(NW, NCH, CH)
    dst = edge_index[1].reshape(NW, NCH, CH)
    p1 = _edge_agg(x.astype(BDT), src, dst)
    h = _relu_sum_mm(p1, W1, BDT)
    p2 = _edge_agg(h, src, dst)
    return _relu_sum_mm(p2, W2, jnp.float32)


# speedup vs baseline: 1.3116x; 1.0283x over previous
import functools

import jax
import jax.numpy as jnp
from jax import lax
from jax.experimental import pallas as pl
from jax.experimental.pallas import tpu as pltpu
from jax.experimental.pallas import tpu_sc as plsc

N = 10000
D = 128
E = 320000

_info = plsc.get_sparse_core_info()
NC = _info.num_cores
NS = _info.num_subcores
NW = NC * NS
CH = 125
NCH = E // NW // CH
GB = 16
NG = NCH // GB
NBUF = 4
RPT = 624
ZR = 16

BDT = jnp.bfloat16

_mesh = plsc.VectorSubcoreMesh(core_axis_name="c", subcore_axis_name="s")


@functools.partial(
    pl.kernel,
    out_type=jax.ShapeDtypeStruct((NC, N, D), BDT),
    mesh=_mesh,
    scratch_types=[
        pltpu.VMEM((2, GB, CH), jnp.int32),
        pltpu.VMEM((2, GB, CH), jnp.int32),
        pltpu.VMEM((NBUF, CH, D), BDT),
        pltpu.VMEM((ZR, D), BDT),
        pltpu.VMEM_SHARED((N, D), BDT),
        [pltpu.SemaphoreType.DMA] * NBUF,
        [pltpu.SemaphoreType.DMA] * NBUF,
        pltpu.SemaphoreType.DMA,
    ],
    compiler_params=pltpu.CompilerParams(use_tc_tiling_on_sc=False),
)
def _edge_agg(h_hbm, src_hbm, dst_hbm, out_hbm, src_b, dst_b, rows, zero_v,
              acc, sem_g, sem_s, sem_i):
    c = lax.axis_index("c")
    s = lax.axis_index("s")
    wid = s * NC + c

    zvec = jnp.zeros((32,), BDT)
    for r in range(ZR):
        for k in range(D // 32):
            zero_v[r, pl.ds(k * 32, 32)] = zvec

    def zero_issue(i, carry):
        pltpu.async_copy(zero_v, acc.at[pl.ds(s * RPT + i * ZR, ZR)], sem_g[0])
        return carry

    lax.fori_loop(0, RPT // ZR, zero_issue, 0)

    @pl.when(s == NS - 1)
    def _():
        pltpu.async_copy(zero_v, acc.at[pl.ds(NS * RPT, ZR)], sem_g[0])

    def zero_drain(i, carry):
        pltpu.make_async_copy(zero_v, acc.at[pl.ds(0, ZR)], sem_g[0]).wait()
        return carry

    lax.fori_loop(0, RPT // ZR, zero_drain, 0)

    @pl.when(s == NS - 1)
    def _():
        pltpu.make_async_copy(zero_v, acc.at[pl.ds(0, ZR)], sem_g[0]).wait()

    plsc.subcore_barrier()

    def _wait_gather(b):
        pltpu.make_async_copy(h_hbm.at[src_b.at[0, 0]], rows.at[b],
                              sem_g[b]).wait()

    def _wait_scatter(b):
        pltpu.make_async_copy(rows.at[b], acc.at[dst_b.at[0, 0]],
                              sem_s[b]).wait()

    def _wait_idx():
        pltpu.make_async_copy(src_hbm.at[0, pl.ds(0, GB)], src_b.at[0],
                              sem_i).wait()

    pltpu.async_copy(src_hbm.at[wid, pl.ds(0, GB)], src_b.at[0], sem_i)
    pltpu.async_copy(dst_hbm.at[wid, pl.ds(0, GB)], dst_b.at[0], sem_i)
    _wait_idx()
    _wait_idx()

    def group_body(k, carry):
        k2 = lax.rem(k, 2)
        kn = lax.rem(k + 1, 2)

        @pl.when(k < NG - 1)
        def _():
            off = pl.multiple_of((k + 1) * GB, 8)
            pltpu.async_copy(src_hbm.at[wid, pl.ds(off, GB)], src_b.at[kn],
                             sem_i)
            pltpu.async_copy(dst_hbm.at[wid, pl.ds(off, GB)], dst_b.at[kn],
                             sem_i)

        pltpu.async_copy(h_hbm.at[src_b.at[k2, 0]], rows.at[0], sem_g[0])
        pltpu.async_copy(h_hbm.at[src_b.at[k2, 1]], rows.at[1], sem_g[1])

        def edge_body(g, carry2):
            for b in range(NBUF):
                j = NBUF * g + b
                b2 = (b + 2) % NBUF
                _wait_gather(b)
                pltpu.async_copy(rows.at[b], acc.at[dst_b.at[k2, j]],
                                 sem_s[b], add=True)

                @pl.when(j >= 2)
                def _():
                    _wait_scatter(b2)

                @pl.when(j <= GB - 3)
                def _():
                    pltpu.async_copy(h_hbm.at[src_b.at[k2, j + 2]],
                                     rows.at[b2], sem_g[b2])
            return carry2

        lax.fori_loop(0, GB // NBUF, edge_body, 0)
        _wait_scatter((GB - 2) % NBUF)
        _wait_scatter((GB - 1) % NBUF)

        @pl.when(k < NG - 1)
        def _():
            _wait_idx()
            _wait_idx()

        return carry

    lax.fori_loop(0, NG, group_body, 0)
    plsc.subcore_barrier()

    pltpu.sync_copy(acc.at[pl.ds(s * RPT, RPT)],
                    out_hbm.at[c, pl.ds(s * RPT, RPT)])

    @pl.when(s == NS - 1)
    def _():
        pltpu.sync_copy(acc.at[pl.ds(NS * RPT, N - NS * RPT)],
                        out_hbm.at[c, pl.ds(NS * RPT, N - NS * RPT)])


_R = 1000


def _relu_sum_mm_kernel(p_ref, w_ref, o_ref):
    h = p_ref[0].astype(jnp.float32) + p_ref[1].astype(jnp.float32)
    r = jnp.maximum(
        jnp.dot(h, w_ref[...], preferred_element_type=jnp.float32,
                precision=jax.lax.Precision.HIGHEST), 0.0)
    o_ref[...] = r.astype(o_ref.dtype)


def _relu_sum_mm(p, W, out_dtype):
    return pl.pallas_call(
        _relu_sum_mm_kernel,
        grid=(N // _R,),
        in_specs=[pl.BlockSpec((NC, _R, D), lambda i: (0, i, 0)),
                  pl.BlockSpec((D, D), lambda i: (0, 0))],
        out_specs=pl.BlockSpec((_R, D), lambda i: (i, 0)),
        out_shape=jax.ShapeDtypeStruct((N, D), out_dtype),
    )(p, W)


def kernel(x, edge_index, W1, W2):
    src = edge_index[0].reshape(NW, NCH, CH)
    dst = edge_index[1].reshape(NW, NCH, CH)
    p1 = _edge_agg(x.astype(BDT), src, dst)
    h = _relu_sum_mm(p1, W1, BDT)
    p2 = _edge_agg(h, src, dst)
    return _relu_sum_mm(p2, W2, jnp.float32)


# speedup vs baseline: 1.3236x; 1.0091x over previous
import functools

import jax
import jax.numpy as jnp
from jax import lax
from jax.experimental import pallas as pl
from jax.experimental.pallas import tpu as pltpu
from jax.experimental.pallas import tpu_sc as plsc

N = 10000
D = 128
E = 320000

_info = plsc.get_sparse_core_info()
NC = _info.num_cores
NS = _info.num_subcores
NW = NC * NS
CH = 125
NCH = E // NW // CH
NBUF = 4
RPT = 624

BDT = jnp.bfloat16

_mesh = plsc.VectorSubcoreMesh(core_axis_name="c", subcore_axis_name="s")


@functools.partial(
    pl.kernel,
    out_type=jax.ShapeDtypeStruct((NC, N, D), BDT),
    mesh=_mesh,
    scratch_types=[
        pltpu.VMEM((NCH, CH), jnp.int32),
        pltpu.VMEM((NCH, CH), jnp.int32),
        pltpu.VMEM((NBUF, CH, D), BDT),
        pltpu.VMEM_SHARED((N, D), BDT),
        [pltpu.SemaphoreType.DMA] * NBUF,
        [pltpu.SemaphoreType.DMA] * NBUF,
        pltpu.SemaphoreType.DMA,
    ],
    compiler_params=pltpu.CompilerParams(use_tc_tiling_on_sc=False),
)
def _edge_agg(h_hbm, src_hbm, dst_hbm, zero_hbm, out_hbm, src_b, dst_b, rows,
              acc, sem_g, sem_s, sem_i):
    c = lax.axis_index("c")
    s = lax.axis_index("s")
    wid = s * NC + c

    pltpu.async_copy(zero_hbm.at[pl.ds(s * RPT, RPT)],
                     acc.at[pl.ds(s * RPT, RPT)], sem_i)
    pltpu.async_copy(src_hbm.at[wid], src_b, sem_i)
    pltpu.async_copy(dst_hbm.at[wid], dst_b, sem_i)

    @pl.when(s == NS - 1)
    def _():
        pltpu.async_copy(zero_hbm.at[pl.ds(NS * RPT, N - NS * RPT)],
                         acc.at[pl.ds(NS * RPT, N - NS * RPT)], sem_i)
        pltpu.make_async_copy(zero_hbm.at[pl.ds(NS * RPT, N - NS * RPT)],
                              acc.at[pl.ds(NS * RPT, N - NS * RPT)],
                              sem_i).wait()

    pltpu.make_async_copy(zero_hbm.at[pl.ds(0, RPT)], acc.at[pl.ds(0, RPT)],
                          sem_i).wait()
    pltpu.make_async_copy(src_hbm.at[0], src_b, sem_i).wait()
    pltpu.make_async_copy(dst_hbm.at[0], dst_b, sem_i).wait()
    plsc.subcore_barrier()

    def _wait_gather(b):
        pltpu.make_async_copy(h_hbm.at[src_b.at[0]], rows.at[b],
                              sem_g[b]).wait()

    def _wait_scatter(b):
        pltpu.make_async_copy(rows.at[b], acc.at[dst_b.at[0]],
                              sem_s[b]).wait()

    pltpu.async_copy(h_hbm.at[src_b.at[0]], rows.at[0], sem_g[0])
    pltpu.async_copy(h_hbm.at[src_b.at[1]], rows.at[1], sem_g[1])

    def edge_body(g, carry):
        for b in range(NBUF):
            j = NBUF * g + b
            b2 = (b + 2) % NBUF
            _wait_gather(b)
            pltpu.async_copy(rows.at[b], acc.at[dst_b.at[j]], sem_s[b],
                             add=True)

            @pl.when(j >= 2)
            def _():
                _wait_scatter(b2)

            @pl.when(j <= NCH - 3)
            def _():
                pltpu.async_copy(h_hbm.at[src_b.at[j + 2]], rows.at[b2],
                                 sem_g[b2])
        return carry

    lax.fori_loop(0, NCH // NBUF, edge_body, 0)
    _wait_scatter((NCH - 2) % NBUF)
    _wait_scatter((NCH - 1) % NBUF)
    plsc.subcore_barrier()

    pltpu.sync_copy(acc.at[pl.ds(s * RPT, RPT)],
                    out_hbm.at[c, pl.ds(s * RPT, RPT)])

    @pl.when(s == NS - 1)
    def _():
        pltpu.sync_copy(acc.at[pl.ds(NS * RPT, N - NS * RPT)],
                        out_hbm.at[c, pl.ds(NS * RPT, N - NS * RPT)])


_R = 1000


def _relu_sum_mm_kernel(p_ref, w_ref, o_ref):
    h = p_ref[0].astype(jnp.float32) + p_ref[1].astype(jnp.float32)
    r = jnp.maximum(
        jnp.dot(h, w_ref[...], preferred_element_type=jnp.float32,
                precision=jax.lax.Precision.HIGHEST), 0.0)
    o_ref[...] = r.astype(o_ref.dtype)


def _relu_sum_mm(p, W, out_dtype):
    return pl.pallas_call(
        _relu_sum_mm_kernel,
        grid=(N // _R,),
        in_specs=[pl.BlockSpec((NC, _R, D), lambda i: (0, i, 0)),
                  pl.BlockSpec((D, D), lambda i: (0, 0))],
        out_specs=pl.BlockSpec((_R, D), lambda i: (i, 0)),
        out_shape=jax.ShapeDtypeStruct((N, D), out_dtype),
    )(p, W)


def kernel(x, edge_index, W1, W2):
    src = edge_index[0].reshape(NW, NCH, CH)
    dst = edge_index[1].reshape(NW, NCH, CH)
    zeros = jnp.zeros((N, D), BDT)
    p1 = _edge_agg(x.astype(BDT), src, dst, zeros)
    h = _relu_sum_mm(p1, W1, BDT)
    p2 = _edge_agg(h, src, dst, zeros)
    return _relu_sum_mm(p2, W2, jnp.float32)


# speedup vs baseline: 1.4018x; 1.0591x over previous
import functools

import jax
import jax.numpy as jnp
from jax import lax
from jax.experimental import pallas as pl
from jax.experimental.pallas import tpu as pltpu
from jax.experimental.pallas import tpu_sc as plsc

N = 10000
D = 128
E = 320000

_info = plsc.get_sparse_core_info()
NC = _info.num_cores
NS = _info.num_subcores
NW = NC * NS
CH = 125
NCH = E // NW // CH
NBUF = 8
AHD = 4
RPT = 624

BDT = jnp.bfloat16

_mesh = plsc.VectorSubcoreMesh(core_axis_name="c", subcore_axis_name="s")


@functools.partial(
    pl.kernel,
    out_type=jax.ShapeDtypeStruct((NC, N, D), BDT),
    mesh=_mesh,
    scratch_types=[
        pltpu.VMEM((NCH, CH), jnp.int32),
        pltpu.VMEM((NCH, CH), jnp.int32),
        pltpu.VMEM((NBUF, CH, D), BDT),
        pltpu.VMEM_SHARED((N, D), BDT),
        [pltpu.SemaphoreType.DMA] * NBUF,
        [pltpu.SemaphoreType.DMA] * NBUF,
        pltpu.SemaphoreType.DMA,
    ],
    compiler_params=pltpu.CompilerParams(use_tc_tiling_on_sc=False),
)
def _edge_agg(h_hbm, src_hbm, dst_hbm, zero_hbm, out_hbm, src_b, dst_b, rows,
              acc, sem_g, sem_s, sem_i):
    c = lax.axis_index("c")
    s = lax.axis_index("s")
    wid = s * NC + c

    pltpu.async_copy(zero_hbm.at[pl.ds(s * RPT, RPT)],
                     acc.at[pl.ds(s * RPT, RPT)], sem_i)
    pltpu.async_copy(src_hbm.at[wid], src_b, sem_i)
    pltpu.async_copy(dst_hbm.at[wid], dst_b, sem_i)

    @pl.when(s == NS - 1)
    def _():
        pltpu.async_copy(zero_hbm.at[pl.ds(NS * RPT, N - NS * RPT)],
                         acc.at[pl.ds(NS * RPT, N - NS * RPT)], sem_i)
        pltpu.make_async_copy(zero_hbm.at[pl.ds(NS * RPT, N - NS * RPT)],
                              acc.at[pl.ds(NS * RPT, N - NS * RPT)],
                              sem_i).wait()

    pltpu.make_async_copy(zero_hbm.at[pl.ds(0, RPT)], acc.at[pl.ds(0, RPT)],
                          sem_i).wait()
    pltpu.make_async_copy(src_hbm.at[0], src_b, sem_i).wait()
    pltpu.make_async_copy(dst_hbm.at[0], dst_b, sem_i).wait()
    plsc.subcore_barrier()

    def _wait_gather(b):
        pltpu.make_async_copy(h_hbm.at[src_b.at[0]], rows.at[b],
                              sem_g[b]).wait()

    def _wait_scatter(b):
        pltpu.make_async_copy(rows.at[b], acc.at[dst_b.at[0]],
                              sem_s[b]).wait()

    for a in range(AHD):
        pltpu.async_copy(h_hbm.at[src_b.at[a]], rows.at[a], sem_g[a])

    def edge_body(g, carry):
        for b in range(NBUF):
            j = NBUF * g + b
            b2 = (b + AHD) % NBUF
            _wait_gather(b)
            pltpu.async_copy(rows.at[b], acc.at[dst_b.at[j]], sem_s[b],
                             add=True)

            @pl.when(j >= NBUF - AHD)
            def _():
                _wait_scatter(b2)

            @pl.when(j <= NCH - AHD - 1)
            def _():
                pltpu.async_copy(h_hbm.at[src_b.at[j + AHD]], rows.at[b2],
                                 sem_g[b2])
        return carry

    lax.fori_loop(0, NCH // NBUF, edge_body, 0)
    for a in range(NBUF - AHD):
        _wait_scatter((NCH - (NBUF - AHD) + a) % NBUF)
    plsc.subcore_barrier()

    pltpu.sync_copy(acc.at[pl.ds(s * RPT, RPT)],
                    out_hbm.at[c, pl.ds(s * RPT, RPT)])

    @pl.when(s == NS - 1)
    def _():
        pltpu.sync_copy(acc.at[pl.ds(NS * RPT, N - NS * RPT)],
                        out_hbm.at[c, pl.ds(NS * RPT, N - NS * RPT)])


_R = 1000


def _relu_sum_mm_kernel(p_ref, w_ref, o_ref):
    h = p_ref[0].astype(jnp.float32) + p_ref[1].astype(jnp.float32)
    r = jnp.maximum(
        jnp.dot(h, w_ref[...], preferred_element_type=jnp.float32,
                precision=jax.lax.Precision.HIGHEST), 0.0)
    o_ref[...] = r.astype(o_ref.dtype)


def _relu_sum_mm(p, W, out_dtype):
    return pl.pallas_call(
        _relu_sum_mm_kernel,
        grid=(N // _R,),
        in_specs=[pl.BlockSpec((NC, _R, D), lambda i: (0, i, 0)),
                  pl.BlockSpec((D, D), lambda i: (0, 0))],
        out_specs=pl.BlockSpec((_R, D), lambda i: (i, 0)),
        out_shape=jax.ShapeDtypeStruct((N, D), out_dtype),
    )(p, W)


def kernel(x, edge_index, W1, W2):
    src = edge_index[0].reshape(NW, NCH, CH)
    dst = edge_index[1].reshape(NW, NCH, CH)
    zeros = jnp.zeros((N, D), BDT)
    p1 = _edge_agg(x.astype(BDT), src, dst, zeros)
    h = _relu_sum_mm(p1, W1, BDT)
    p2 = _edge_agg(h, src, dst, zeros)
    return _relu_sum_mm(p2, W2, jnp.float32)
